# baseline (speedup 1.0000x reference)
import functools

import jax
import jax.numpy as jnp
from jax.experimental import pallas as pl
from jax.experimental.pallas import tpu as pltpu


def _round_up(x, m):
    return (x + m - 1) // m * m


def _lse_m_kernel(table_ref, m_ref):
    t = table_ref[...]
    mx = jnp.max(t, axis=1, keepdims=True)
    lse = jnp.log(jnp.sum(jnp.exp(t - mx), axis=1, keepdims=True)) + mx
    m_ref[...] = lse - t


def _onehot_t(tok_row, vpad, tblk):
    viota = jax.lax.broadcasted_iota(jnp.int32, (vpad, tblk), 0)
    return jnp.where(tok_row == viota, 1.0, 0.0).astype(jnp.bfloat16)


def _loss_tile_kernel(idx_ref, tgt_ref, table_ref, m_ref,
                      logits_ref, partial_ref):
    tblk, vpad = logits_ref.shape
    tok = idx_ref[...].reshape(1, tblk)
    tgt = tgt_ref[...].reshape(1, tblk)
    p = _onehot_t(tok, vpad, tblk)
    q = _onehot_t(tgt, vpad, tblk)

    logits_ref[...] = jax.lax.dot_general(
        p, table_ref[...], (((0,), (0,)), ((), ())),
        preferred_element_type=jnp.float32)

    c = jax.lax.dot_general(
        p, q, (((1,), (1,)), ((), ())),
        preferred_element_type=jnp.float32)
    partial = jnp.sum(c * m_ref[...], axis=0, keepdims=True)
    partial_ref[...] = partial.reshape(1, 1, vpad)


def _logits_tile_kernel(idx_ref, table_ref, logits_ref):
    tblk, vpad = logits_ref.shape
    tok = idx_ref[...].reshape(1, tblk)
    p = _onehot_t(tok, vpad, tblk)
    logits_ref[...] = jax.lax.dot_general(
        p, table_ref[...], (((0,), (0,)), ((), ())),
        preferred_element_type=jnp.float32)


def _make_stream_kernel(t2, tblk, ch, nslot, vpad):
    s_chunks = tblk // ch

    def _stream_kernel(idx_ref, tgt_ref, table_ref, m_ref,
                       logits_ref, partial_ref, buf_ref, sem_ref):
        c_id = pl.program_id(0)
        j = pl.program_id(1)
        tile = c_id * t2 + j
        acc = jnp.zeros((1, vpad), jnp.float32)
        for s in range(s_chunks):
            slot = s % nslot
            if s >= nslot:
                pltpu.make_async_copy(
                    buf_ref.at[slot], buf_ref.at[slot],
                    sem_ref.at[slot]).wait()
            else:
                @pl.when(j > 0)
                def _wait_prev():
                    pltpu.make_async_copy(
                        buf_ref.at[slot], buf_ref.at[slot],
                        sem_ref.at[slot]).wait()

            tok = idx_ref[:, :, pl.ds(s * ch, ch)].reshape(1, ch)
            tgt = tgt_ref[:, :, pl.ds(s * ch, ch)].reshape(1, ch)
            p = _onehot_t(tok, vpad, ch)
            q = _onehot_t(tgt, vpad, ch)
            buf_ref[slot] = jax.lax.dot_general(
                p, table_ref[...], (((0,), (0,)), ((), ())),
                preferred_element_type=jnp.float32)
            cmat = jax.lax.dot_general(
                p, q, (((1,), (1,)), ((), ())),
                preferred_element_type=jnp.float32)
            acc = acc + jnp.sum(cmat * m_ref[...], axis=0, keepdims=True)

            pltpu.make_async_copy(
                buf_ref.at[slot],
                logits_ref.at[pl.ds(tile * tblk + s * ch, ch), :],
                sem_ref.at[slot]).start()

        partial_ref[...] = acc.reshape(1, 1, vpad)

        @pl.when(j == t2 - 1)
        def _drain():
            for slot in range(min(nslot, s_chunks)):
                pltpu.make_async_copy(
                    buf_ref.at[slot], buf_ref.at[slot],
                    sem_ref.at[slot]).wait()

    return _stream_kernel


@functools.partial(jax.jit, static_argnames=("tblk",))
def _forward(idx, targets, table, *, tblk=32768):
    B, T = idx.shape
    V = table.shape[0]
    N = B * T

    Vpad = _round_up(V, 128)
    CH = 4096
    NSLOT = 4
    has_targets = targets is not None
    use_stream = has_targets and N % tblk == 0 and tblk % CH == 0 \
        and (N // tblk) % 2 == 0 and tblk // CH >= NSLOT

    table_f32 = table.astype(jnp.float32)
    table_pad = jnp.pad(table_f32, ((0, Vpad - V), (0, Vpad - V)))
    if has_targets and Vpad > V:
        table_pad = table_pad.at[:, V:].set(jnp.float32(-1e30))
    table_bf16 = table_pad.astype(jnp.bfloat16)

    if has_targets:
        m_mat = pl.pallas_call(
            _lse_m_kernel,
            out_shape=jax.ShapeDtypeStruct((Vpad, Vpad), jnp.float32),
        )(table_pad)

    if use_stream:
        TBLK = tblk
        num_tiles = N // TBLK
        T2 = num_tiles // 2
        idx3 = idx.reshape(num_tiles, 1, TBLK).astype(jnp.int32)
        tgt3 = targets.reshape(num_tiles, 1, TBLK).astype(jnp.int32)

        tok_spec = pl.BlockSpec((1, 1, TBLK), lambda c, j: (c * T2 + j, 0, 0))
        small_spec = pl.BlockSpec((Vpad, Vpad), lambda c, j: (0, 0))
        cparams = pltpu.CompilerParams(
            dimension_semantics=("parallel", "arbitrary"),
            vmem_limit_bytes=60 * 1024 * 1024)

        logits, partials = pl.pallas_call(
            _make_stream_kernel(T2, TBLK, CH, NSLOT, Vpad),
            out_shape=(
                jax.ShapeDtypeStruct((N, Vpad), jnp.float32),
                jax.ShapeDtypeStruct((num_tiles, 1, Vpad), jnp.float32),
            ),
            grid_spec=pltpu.PrefetchScalarGridSpec(
                num_scalar_prefetch=0,
                grid=(2, T2),
                in_specs=[tok_spec, tok_spec, small_spec, small_spec],
                out_specs=[pl.BlockSpec(memory_space=pl.ANY),
                           pl.BlockSpec((1, 1, Vpad),
                                        lambda c, j: (c * T2 + j, 0, 0))],
                scratch_shapes=[
                    pltpu.VMEM((NSLOT, CH, Vpad), jnp.float32),
                    pltpu.SemaphoreType.DMA((NSLOT,)),
                ],
            ),
            compiler_params=cparams,
        )(idx3, tgt3, table_bf16, m_mat)

        loss = jnp.sum(partials) / jnp.float32(N)
        logits_flat = logits[:, :V] if Vpad > V else logits
        return logits_flat, loss

    TBLK = min(16384, _round_up(N, 128))
    Npad = _round_up(N, TBLK)
    num_tiles = Npad // TBLK

    idx_flat = idx.reshape(-1).astype(jnp.int32)
    if Npad > N:
        idx_flat = jnp.pad(idx_flat, (0, Npad - N))
    idx3 = idx_flat.reshape(num_tiles, 1, TBLK)

    cparams = pltpu.CompilerParams(
        dimension_semantics=("parallel",),
        vmem_limit_bytes=60 * 1024 * 1024)
    tok_spec = pl.BlockSpec((1, 1, TBLK), lambda i: (i, 0, 0))
    table_spec = pl.BlockSpec((Vpad, Vpad), lambda i: (0, 0))
    logits_spec = pl.BlockSpec((TBLK, Vpad), lambda i: (i, 0))

    if has_targets:
        tgt_flat = targets.reshape(-1).astype(jnp.int32)
        if Npad > N:
            tgt_flat = jnp.pad(tgt_flat, (0, Npad - N))
        tgt3 = tgt_flat.reshape(num_tiles, 1, TBLK)

        logits_pad, partials = pl.pallas_call(
            _loss_tile_kernel,
            out_shape=(
                jax.ShapeDtypeStruct((Npad, Vpad), jnp.float32),
                jax.ShapeDtypeStruct((num_tiles, 1, Vpad), jnp.float32),
            ),
            grid_spec=pltpu.PrefetchScalarGridSpec(
                num_scalar_prefetch=0,
                grid=(num_tiles,),
                in_specs=[tok_spec, tok_spec, table_spec,
                          pl.BlockSpec((Vpad, Vpad), lambda i: (0, 0))],
                out_specs=[logits_spec,
                           pl.BlockSpec((1, 1, Vpad), lambda i: (i, 0, 0))],
            ),
            compiler_params=cparams,
        )(idx3, tgt3, table_bf16, m_mat)

        loss_sum = jnp.sum(partials)
        if Npad > N:
            loss_sum = loss_sum - jnp.float32(Npad - N) * m_mat[0, 0]
        loss = loss_sum / jnp.float32(N)
        logits_flat = logits_pad[:N, :V] if (Npad > N or Vpad > V) \
            else logits_pad
        return logits_flat, loss

    logits_pad = pl.pallas_call(
        _logits_tile_kernel,
        out_shape=jax.ShapeDtypeStruct((Npad, Vpad), jnp.float32),
        grid_spec=pltpu.PrefetchScalarGridSpec(
            num_scalar_prefetch=0,
            grid=(num_tiles,),
            in_specs=[tok_spec, table_spec],
            out_specs=logits_spec,
        ),
        compiler_params=cparams,
    )(idx3, table_bf16)
    if Npad > N or Vpad > V:
        logits_pad = logits_pad[:N, :V]
    return logits_pad.reshape(B, T, V), None


def kernel(idx, targets, table):
    return _forward(idx, targets, table)


# speedup vs baseline: 1.0391x; 1.0391x over previous
import functools

import jax
import jax.numpy as jnp
from jax.experimental import pallas as pl
from jax.experimental.pallas import tpu as pltpu


def _round_up(x, m):
    return (x + m - 1) // m * m


def _lse_m_kernel(table_ref, m_ref):
    t = table_ref[...]
    mx = jnp.max(t, axis=1, keepdims=True)
    lse = jnp.log(jnp.sum(jnp.exp(t - mx), axis=1, keepdims=True)) + mx
    m_ref[...] = lse - t


def _onehot_t(tok_row, vpad, tblk):
    viota = jax.lax.broadcasted_iota(jnp.int32, (vpad, tblk), 0)
    return jnp.where(tok_row == viota, 1.0, 0.0).astype(jnp.bfloat16)


def _loss_tile_kernel(idx_ref, tgt_ref, table_ref, m_ref,
                      logits_ref, partial_ref):
    tblk, vpad = logits_ref.shape
    tok = idx_ref[...].reshape(1, tblk)
    tgt = tgt_ref[...].reshape(1, tblk)
    p = _onehot_t(tok, vpad, tblk)
    q = _onehot_t(tgt, vpad, tblk)

    logits_ref[...] = jax.lax.dot_general(
        p, table_ref[...], (((0,), (0,)), ((), ())),
        preferred_element_type=jnp.float32)

    c = jax.lax.dot_general(
        p, q, (((1,), (1,)), ((), ())),
        preferred_element_type=jnp.float32)
    partial = jnp.sum(c * m_ref[...], axis=0, keepdims=True)
    partial_ref[...] = partial.reshape(1, 1, vpad)


def _logits_tile_kernel(idx_ref, table_ref, logits_ref):
    tblk, vpad = logits_ref.shape
    tok = idx_ref[...].reshape(1, tblk)
    p = _onehot_t(tok, vpad, tblk)
    logits_ref[...] = jax.lax.dot_general(
        p, table_ref[...], (((0,), (0,)), ((), ())),
        preferred_element_type=jnp.float32)


def _make_stream_kernel(t2, tblk, ch, nslot, vpad):
    s_chunks = tblk // ch

    def _stream_kernel(idx_ref, tgt_ref, table_ref, m_ref,
                       logits_ref, partial_ref, buf_ref, sem_ref):
        c_id = pl.program_id(0)
        j = pl.program_id(1)
        tile = c_id * t2 + j
        acc = jnp.zeros((1, vpad), jnp.float32)
        for s in range(s_chunks):
            slot = s % nslot
            if s >= nslot:
                pltpu.make_async_copy(
                    buf_ref.at[slot], buf_ref.at[slot],
                    sem_ref.at[slot]).wait()
            else:
                @pl.when(j > 0)
                def _wait_prev():
                    pltpu.make_async_copy(
                        buf_ref.at[slot], buf_ref.at[slot],
                        sem_ref.at[slot]).wait()

            tok = idx_ref[:, :, pl.ds(s * ch, ch)].reshape(1, ch)
            tgt = tgt_ref[:, :, pl.ds(s * ch, ch)].reshape(1, ch)
            p = _onehot_t(tok, vpad, ch)
            q = _onehot_t(tgt, vpad, ch)
            buf_ref[slot] = jax.lax.dot_general(
                p, table_ref[...], (((0,), (0,)), ((), ())),
                preferred_element_type=jnp.float32)
            cmat = jax.lax.dot_general(
                p, q, (((1,), (1,)), ((), ())),
                preferred_element_type=jnp.float32)
            acc = acc + jnp.sum(cmat * m_ref[...], axis=0, keepdims=True)

            pltpu.make_async_copy(
                buf_ref.at[slot],
                logits_ref.at[pl.ds(tile * tblk + s * ch, ch), :],
                sem_ref.at[slot]).start()

        partial_ref[...] = acc.reshape(1, 1, vpad)

        @pl.when(j == t2 - 1)
        def _drain():
            for slot in range(min(nslot, s_chunks)):
                pltpu.make_async_copy(
                    buf_ref.at[slot], buf_ref.at[slot],
                    sem_ref.at[slot]).wait()

    return _stream_kernel


@functools.partial(jax.jit, static_argnames=("tblk",))
def _forward(idx, targets, table, *, tblk=32768):
    B, T = idx.shape
    V = table.shape[0]
    N = B * T

    Vpad = _round_up(V, 128)
    CH = 8192
    NSLOT = 4
    has_targets = targets is not None
    use_stream = has_targets and N % tblk == 0 and tblk % CH == 0 \
        and (N // tblk) % 2 == 0 and tblk // CH >= NSLOT

    table_f32 = table.astype(jnp.float32)
    table_pad = jnp.pad(table_f32, ((0, Vpad - V), (0, Vpad - V)))
    if has_targets and Vpad > V:
        table_pad = table_pad.at[:, V:].set(jnp.float32(-1e30))
    table_bf16 = table_pad.astype(jnp.bfloat16)

    if has_targets:
        m_mat = pl.pallas_call(
            _lse_m_kernel,
            out_shape=jax.ShapeDtypeStruct((Vpad, Vpad), jnp.float32),
        )(table_pad)

    if use_stream:
        TBLK = tblk
        num_tiles = N // TBLK
        T2 = num_tiles // 2
        idx3 = idx.reshape(num_tiles, 1, TBLK).astype(jnp.int32)
        tgt3 = targets.reshape(num_tiles, 1, TBLK).astype(jnp.int32)

        tok_spec = pl.BlockSpec((1, 1, TBLK), lambda c, j: (c * T2 + j, 0, 0))
        small_spec = pl.BlockSpec((Vpad, Vpad), lambda c, j: (0, 0))
        cparams = pltpu.CompilerParams(
            dimension_semantics=("parallel", "arbitrary"),
            vmem_limit_bytes=60 * 1024 * 1024)

        logits, partials = pl.pallas_call(
            _make_stream_kernel(T2, TBLK, CH, NSLOT, Vpad),
            out_shape=(
                jax.ShapeDtypeStruct((N, Vpad), jnp.float32),
                jax.ShapeDtypeStruct((num_tiles, 1, Vpad), jnp.float32),
            ),
            grid_spec=pltpu.PrefetchScalarGridSpec(
                num_scalar_prefetch=0,
                grid=(2, T2),
                in_specs=[tok_spec, tok_spec, small_spec, small_spec],
                out_specs=[pl.BlockSpec(memory_space=pl.ANY),
                           pl.BlockSpec((1, 1, Vpad),
                                        lambda c, j: (c * T2 + j, 0, 0))],
                scratch_shapes=[
                    pltpu.VMEM((NSLOT, CH, Vpad), jnp.float32),
                    pltpu.SemaphoreType.DMA((NSLOT,)),
                ],
            ),
            compiler_params=cparams,
        )(idx3, tgt3, table_bf16, m_mat)

        loss = jnp.sum(partials) / jnp.float32(N)
        logits_flat = logits[:, :V] if Vpad > V else logits
        return logits_flat, loss

    TBLK = min(16384, _round_up(N, 128))
    Npad = _round_up(N, TBLK)
    num_tiles = Npad // TBLK

    idx_flat = idx.reshape(-1).astype(jnp.int32)
    if Npad > N:
        idx_flat = jnp.pad(idx_flat, (0, Npad - N))
    idx3 = idx_flat.reshape(num_tiles, 1, TBLK)

    cparams = pltpu.CompilerParams(
        dimension_semantics=("parallel",),
        vmem_limit_bytes=60 * 1024 * 1024)
    tok_spec = pl.BlockSpec((1, 1, TBLK), lambda i: (i, 0, 0))
    table_spec = pl.BlockSpec((Vpad, Vpad), lambda i: (0, 0))
    logits_spec = pl.BlockSpec((TBLK, Vpad), lambda i: (i, 0))

    if has_targets:
        tgt_flat = targets.reshape(-1).astype(jnp.int32)
        if Npad > N:
            tgt_flat = jnp.pad(tgt_flat, (0, Npad - N))
        tgt3 = tgt_flat.reshape(num_tiles, 1, TBLK)

        logits_pad, partials = pl.pallas_call(
            _loss_tile_kernel,
            out_shape=(
                jax.ShapeDtypeStruct((Npad, Vpad), jnp.float32),
                jax.ShapeDtypeStruct((num_tiles, 1, Vpad), jnp.float32),
            ),
            grid_spec=pltpu.PrefetchScalarGridSpec(
                num_scalar_prefetch=0,
                grid=(num_tiles,),
                in_specs=[tok_spec, tok_spec, table_spec,
                          pl.BlockSpec((Vpad, Vpad), lambda i: (0, 0))],
                out_specs=[logits_spec,
                           pl.BlockSpec((1, 1, Vpad), lambda i: (i, 0, 0))],
            ),
            compiler_params=cparams,
        )(idx3, tgt3, table_bf16, m_mat)

        loss_sum = jnp.sum(partials)
        if Npad > N:
            loss_sum = loss_sum - jnp.float32(Npad - N) * m_mat[0, 0]
        loss = loss_sum / jnp.float32(N)
        logits_flat = logits_pad[:N, :V] if (Npad > N or Vpad > V) \
            else logits_pad
        return logits_flat, loss

    logits_pad = pl.pallas_call(
        _logits_tile_kernel,
        out_shape=jax.ShapeDtypeStruct((Npad, Vpad), jnp.float32),
        grid_spec=pltpu.PrefetchScalarGridSpec(
            num_scalar_prefetch=0,
            grid=(num_tiles,),
            in_specs=[tok_spec, table_spec],
            out_specs=logits_spec,
        ),
        compiler_params=cparams,
    )(idx3, table_bf16)
    if Npad > N or Vpad > V:
        logits_pad = logits_pad[:N, :V]
    return logits_pad.reshape(B, T, V), None


def kernel(idx, targets, table):
    return _forward(idx, targets, table)


# speedup vs baseline: 1.0586x; 1.0188x over previous
import functools

import jax
import jax.numpy as jnp
from jax.experimental import pallas as pl
from jax.experimental.pallas import tpu as pltpu


def _round_up(x, m):
    return (x + m - 1) // m * m


def _lse_m_kernel(table_ref, m_ref):
    t = table_ref[...]
    mx = jnp.max(t, axis=1, keepdims=True)
    lse = jnp.log(jnp.sum(jnp.exp(t - mx), axis=1, keepdims=True)) + mx
    m_ref[...] = lse - t


def _onehot_t(tok_row, vpad, tblk):
    viota = jax.lax.broadcasted_iota(jnp.int32, (vpad, tblk), 0)
    return jnp.where(tok_row == viota, 1.0, 0.0).astype(jnp.bfloat16)


def _loss_tile_kernel(idx_ref, tgt_ref, table_ref, m_ref,
                      logits_ref, partial_ref):
    tblk, vpad = logits_ref.shape
    tok = idx_ref[...].reshape(1, tblk)
    tgt = tgt_ref[...].reshape(1, tblk)
    p = _onehot_t(tok, vpad, tblk)
    q = _onehot_t(tgt, vpad, tblk)

    logits_ref[...] = jax.lax.dot_general(
        p, table_ref[...], (((0,), (0,)), ((), ())),
        preferred_element_type=jnp.float32)

    c = jax.lax.dot_general(
        p, q, (((1,), (1,)), ((), ())),
        preferred_element_type=jnp.float32)
    partial = jnp.sum(c * m_ref[...], axis=0, keepdims=True)
    partial_ref[...] = partial.reshape(1, 1, vpad)


def _logits_tile_kernel(idx_ref, table_ref, logits_ref):
    tblk, vpad = logits_ref.shape
    tok = idx_ref[...].reshape(1, tblk)
    p = _onehot_t(tok, vpad, tblk)
    logits_ref[...] = jax.lax.dot_general(
        p, table_ref[...], (((0,), (0,)), ((), ())),
        preferred_element_type=jnp.float32)


def _make_stream_kernel(t2, tblk, ch, nslot, vpad):
    s_chunks = tblk // ch

    def _stream_kernel(idx_ref, tgt_ref, table_ref, m_ref,
                       logits_ref, partial_ref, buf_ref, sem_ref):
        c_id = pl.program_id(0)
        j = pl.program_id(1)
        tile = c_id * t2 + j
        acc = jnp.zeros((1, vpad), jnp.float32)
        for s in range(s_chunks):
            slot = s % nslot
            if s >= nslot:
                pltpu.make_async_copy(
                    buf_ref.at[slot], buf_ref.at[slot],
                    sem_ref.at[slot]).wait()
            else:
                @pl.when(j > 0)
                def _wait_prev():
                    pltpu.make_async_copy(
                        buf_ref.at[slot], buf_ref.at[slot],
                        sem_ref.at[slot]).wait()

            tok = idx_ref[:, :, pl.ds(s * ch, ch)].reshape(1, ch)
            tgt = tgt_ref[:, :, pl.ds(s * ch, ch)].reshape(1, ch)
            p = _onehot_t(tok, vpad, ch)
            q = _onehot_t(tgt, vpad, ch)
            buf_ref[slot] = jax.lax.dot_general(
                p, table_ref[...], (((0,), (0,)), ((), ())),
                preferred_element_type=jnp.float32)
            cmat = jax.lax.dot_general(
                p, q, (((1,), (1,)), ((), ())),
                preferred_element_type=jnp.float32)
            acc = acc + jnp.sum(cmat * m_ref[...], axis=0, keepdims=True)

            pltpu.make_async_copy(
                buf_ref.at[slot],
                logits_ref.at[pl.ds(tile * tblk + s * ch, ch), :],
                sem_ref.at[slot]).start()

        partial_ref[...] = acc.reshape(1, 1, vpad)

        @pl.when(j == t2 - 1)
        def _drain():
            for slot in range(min(nslot, s_chunks)):
                pltpu.make_async_copy(
                    buf_ref.at[slot], buf_ref.at[slot],
                    sem_ref.at[slot]).wait()

    return _stream_kernel


@functools.partial(jax.jit, static_argnames=("tblk",))
def _forward(idx, targets, table, *, tblk=65536):
    B, T = idx.shape
    V = table.shape[0]
    N = B * T

    Vpad = _round_up(V, 128)
    CH = 8192
    NSLOT = 4
    has_targets = targets is not None
    use_stream = has_targets and N % tblk == 0 and tblk % CH == 0 \
        and (N // tblk) % 2 == 0 and tblk // CH >= NSLOT

    table_f32 = table.astype(jnp.float32)
    table_pad = jnp.pad(table_f32, ((0, Vpad - V), (0, Vpad - V)))
    if has_targets and Vpad > V:
        table_pad = table_pad.at[:, V:].set(jnp.float32(-1e30))
    table_bf16 = table_pad.astype(jnp.bfloat16)

    if has_targets:
        m_mat = pl.pallas_call(
            _lse_m_kernel,
            out_shape=jax.ShapeDtypeStruct((Vpad, Vpad), jnp.float32),
        )(table_pad)

    if use_stream:
        TBLK = tblk
        num_tiles = N // TBLK
        T2 = num_tiles // 2
        idx3 = idx.reshape(num_tiles, 1, TBLK).astype(jnp.int32)
        tgt3 = targets.reshape(num_tiles, 1, TBLK).astype(jnp.int32)

        tok_spec = pl.BlockSpec((1, 1, TBLK), lambda c, j: (c * T2 + j, 0, 0))
        small_spec = pl.BlockSpec((Vpad, Vpad), lambda c, j: (0, 0))
        cparams = pltpu.CompilerParams(
            dimension_semantics=("parallel", "arbitrary"),
            vmem_limit_bytes=60 * 1024 * 1024)

        logits, partials = pl.pallas_call(
            _make_stream_kernel(T2, TBLK, CH, NSLOT, Vpad),
            out_shape=(
                jax.ShapeDtypeStruct((N, Vpad), jnp.float32),
                jax.ShapeDtypeStruct((num_tiles, 1, Vpad), jnp.float32),
            ),
            grid_spec=pltpu.PrefetchScalarGridSpec(
                num_scalar_prefetch=0,
                grid=(2, T2),
                in_specs=[tok_spec, tok_spec, small_spec, small_spec],
                out_specs=[pl.BlockSpec(memory_space=pl.ANY),
                           pl.BlockSpec((1, 1, Vpad),
                                        lambda c, j: (c * T2 + j, 0, 0))],
                scratch_shapes=[
                    pltpu.VMEM((NSLOT, CH, Vpad), jnp.float32),
                    pltpu.SemaphoreType.DMA((NSLOT,)),
                ],
            ),
            compiler_params=cparams,
        )(idx3, tgt3, table_bf16, m_mat)

        loss = jnp.sum(partials) / jnp.float32(N)
        logits_flat = logits[:, :V] if Vpad > V else logits
        return logits_flat, loss

    TBLK = min(16384, _round_up(N, 128))
    Npad = _round_up(N, TBLK)
    num_tiles = Npad // TBLK

    idx_flat = idx.reshape(-1).astype(jnp.int32)
    if Npad > N:
        idx_flat = jnp.pad(idx_flat, (0, Npad - N))
    idx3 = idx_flat.reshape(num_tiles, 1, TBLK)

    cparams = pltpu.CompilerParams(
        dimension_semantics=("parallel",),
        vmem_limit_bytes=60 * 1024 * 1024)
    tok_spec = pl.BlockSpec((1, 1, TBLK), lambda i: (i, 0, 0))
    table_spec = pl.BlockSpec((Vpad, Vpad), lambda i: (0, 0))
    logits_spec = pl.BlockSpec((TBLK, Vpad), lambda i: (i, 0))

    if has_targets:
        tgt_flat = targets.reshape(-1).astype(jnp.int32)
        if Npad > N:
            tgt_flat = jnp.pad(tgt_flat, (0, Npad - N))
        tgt3 = tgt_flat.reshape(num_tiles, 1, TBLK)

        logits_pad, partials = pl.pallas_call(
            _loss_tile_kernel,
            out_shape=(
                jax.ShapeDtypeStruct((Npad, Vpad), jnp.float32),
                jax.ShapeDtypeStruct((num_tiles, 1, Vpad), jnp.float32),
            ),
            grid_spec=pltpu.PrefetchScalarGridSpec(
                num_scalar_prefetch=0,
                grid=(num_tiles,),
                in_specs=[tok_spec, tok_spec, table_spec,
                          pl.BlockSpec((Vpad, Vpad), lambda i: (0, 0))],
                out_specs=[logits_spec,
                           pl.BlockSpec((1, 1, Vpad), lambda i: (i, 0, 0))],
            ),
            compiler_params=cparams,
        )(idx3, tgt3, table_bf16, m_mat)

        loss_sum = jnp.sum(partials)
        if Npad > N:
            loss_sum = loss_sum - jnp.float32(Npad - N) * m_mat[0, 0]
        loss = loss_sum / jnp.float32(N)
        logits_flat = logits_pad[:N, :V] if (Npad > N or Vpad > V) \
            else logits_pad
        return logits_flat, loss

    logits_pad = pl.pallas_call(
        _logits_tile_kernel,
        out_shape=jax.ShapeDtypeStruct((Npad, Vpad), jnp.float32),
        grid_spec=pltpu.PrefetchScalarGridSpec(
            num_scalar_prefetch=0,
            grid=(num_tiles,),
            in_specs=[tok_spec, table_spec],
            out_specs=logits_spec,
        ),
        compiler_params=cparams,
    )(idx3, table_bf16)
    if Npad > N or Vpad > V:
        logits_pad = logits_pad[:N, :V]
    return logits_pad.reshape(B, T, V), None


def kernel(idx, targets, table):
    return _forward(idx, targets, table)
